# baseline (speedup 1.0000x reference)
import functools

import jax
import jax.numpy as jnp
from jax import lax
from jax.experimental import pallas as pl
from jax.experimental.pallas import tpu as pltpu
from jax.experimental.pallas import tpu_sc as plsc

_BEV_FEAT_SHAPE = (1, 1, 200, 200, 64)
_NC, _NS, _LANES = 2, 16, 16
_NW = _NC * _NS
_IC = 80
_CC = 50
_NBUF = 3


@functools.lru_cache(maxsize=None)
def _make_bevpool(n_cells, mx, C, p1, f1):
    del p1, f1
    kc = C // _LANES
    G = _CC * mx
    n_str = G // _IC
    cells_per_w = n_cells // _NW
    n_chunks = cells_per_w // _CC
    idx_rows = cells_per_w * mx // _IC

    mesh = plsc.VectorSubcoreMesh(
        core_axis_name="c", subcore_axis_name="s",
        num_cores=_NC, num_subcores=_NS)

    del idx_rows
    n_idx = cells_per_w * mx
    scratch = ([pltpu.VMEM((n_idx,), jnp.int32)] * 2
               + [pltpu.VMEM((G,), jnp.float32)] * _NBUF
               + [pltpu.VMEM((G, C), jnp.bfloat16)] * _NBUF
               + [pltpu.VMEM((_CC * C,), jnp.bfloat16)] * _NBUF
               + [pltpu.SemaphoreType.DMA] * 2)

    @functools.partial(
        pl.kernel,
        out_type=jax.ShapeDtypeStruct((n_cells * C,), jnp.bfloat16),
        mesh=mesh,
        compiler_params=pltpu.CompilerParams(
            use_tc_tiling_on_sc=False, needs_layout_passes=False),
        scratch_types=scratch,
    )
    def bev_kernel(depth_hbm, feat_hbm, rd_hbm, rf_hbm, out_hbm,
                   rdx_v, rfx_v, dv0, dv1, dv2, fr0, fr1, fr2,
                   ob0, ob1, ob2, sem_g, sem_o):
        dvs = (dv0, dv1, dv2)
        frs = (fr0, fr1, fr2)
        obs = (ob0, ob1, ob2)
        wid = lax.axis_index("s") * _NC + lax.axis_index("c")
        cell_base = wid * cells_per_w
        idx_base = wid * n_idx

        pltpu.sync_copy(rd_hbm.at[pl.ds(idx_base, n_idx)], rdx_v)
        pltpu.sync_copy(rf_hbm.at[pl.ds(idx_base, n_idx)], rfx_v)

        def fire_gathers(t, dv, fr):
            for i in range(n_str):
                r = (t * n_str + i) * _IC
                pltpu.async_copy(
                    depth_hbm.at[rdx_v.at[pl.ds(r, _IC)]],
                    dv.at[pl.ds(i * _IC, _IC)], sem_g)
                pltpu.async_copy(
                    feat_hbm.at[rfx_v.at[pl.ds(r, _IC)]],
                    fr.at[pl.ds(i * _IC, _IC)], sem_g)

        def drain_gathers(dv, fr):
            for i in range(n_str):
                pltpu.make_async_copy(
                    depth_hbm.at[rdx_v.at[pl.ds(i * _IC, _IC)]],
                    dv.at[pl.ds(i * _IC, _IC)], sem_g).wait()
                pltpu.make_async_copy(
                    feat_hbm.at[rfx_v.at[pl.ds(i * _IC, _IC)]],
                    fr.at[pl.ds(i * _IC, _IC)], sem_g).wait()

        def compute_chunk(t, dv, fr, ob):
            def cell_body(j, _):
                pb = j * mx
                dvec = dv[pl.ds(pb, mx)]
                dd = plsc.bitcast(
                    plsc.pack(dvec, dvec, format=plsc.PackFormat.INTERLEAVED),
                    jnp.int32)
                acc = [jnp.zeros((_LANES,), jnp.float32) for _ in range(kc)]
                for p in range(mx):
                    dsplat = plsc.bitcast(
                        jnp.broadcast_to(dd[p], (_LANES,)), jnp.bfloat16)
                    for k2 in range(kc // 2):
                        packed = fr[pb + p, pl.ds(k2 * 2 * _LANES, 2 * _LANES)]
                        lo, hi = plsc.unpack(
                            dsplat * packed, format=plsc.PackFormat.INTERLEAVED)
                        acc[2 * k2] = acc[2 * k2] + lo
                        acc[2 * k2 + 1] = acc[2 * k2 + 1] + hi
                for k2 in range(kc // 2):
                    ob[pl.ds(j * C + k2 * 2 * _LANES, 2 * _LANES)] = plsc.pack(
                        acc[2 * k2], acc[2 * k2 + 1],
                        format=plsc.PackFormat.INTERLEAVED)
                return 0

            lax.fori_loop(0, _CC, cell_body, 0)
            pltpu.async_copy(
                ob, out_hbm.at[pl.ds((cell_base + t * _CC) * C, _CC * C)],
                sem_o)

        def drain_out(ob):
            pltpu.make_async_copy(
                ob, out_hbm.at[pl.ds(cell_base * C, _CC * C)], sem_o).wait()

        fire_gathers(0, dvs[0], frs[0])
        fire_gathers(1, dvs[1], frs[1])

        n_main = n_chunks - 4

        def ring_body(q, _):
            for u in range(_NBUF):
                t = _NBUF * q + u
                fire_gathers(t + 2, dvs[(u + 2) % _NBUF], frs[(u + 2) % _NBUF])
                drain_gathers(dvs[u], frs[u])

                @pl.when(t >= _NBUF)
                def _():
                    drain_out(obs[u])

                compute_chunk(t, dvs[u], frs[u], obs[u])
            return 0

        lax.fori_loop(0, n_main // _NBUF, ring_body, 0)
        for tt in range(n_main, n_chunks):
            u = tt % _NBUF
            if tt + 2 < n_chunks:
                fire_gathers(tt + 2, dvs[(u + 2) % _NBUF], frs[(u + 2) % _NBUF])
            drain_gathers(dvs[u], frs[u])
            drain_out(obs[u])
            compute_chunk(tt, dvs[u], frs[u], obs[u])
        for tt in range(n_chunks - _NBUF, n_chunks):
            drain_out(obs[tt % _NBUF])

    return bev_kernel


def kernel(depth, feat, ranks_depth, ranks_feat, maxn):
    del maxn
    C = feat.shape[-1]
    _, oD, oW, oH, _ = _BEV_FEAT_SHAPE
    n_cells = oD * oW * oH
    L = ranks_depth.shape[0]
    mx = L // n_cells
    npad = 128 - depth.size % 128
    depth_flat = jnp.concatenate(
        [depth.reshape(-1), jnp.zeros((npad,), jnp.float32)])
    feat_2d = jnp.concatenate(
        [feat.reshape(-1, C), jnp.zeros((1, C), jnp.float32)], axis=0)
    f1 = feat_2d.shape[0]
    feat_bf = (feat_2d.reshape(f1, C // 32, 2, 16)
               .swapaxes(2, 3).reshape(f1, C).astype(jnp.bfloat16))
    fn = _make_bevpool(n_cells, mx, C, depth_flat.shape[0], f1)
    out = fn(depth_flat, feat_bf, ranks_depth, ranks_feat)
    out = (out.reshape(n_cells, C // 32, 16, 2).swapaxes(2, 3)
           .astype(jnp.float32).reshape(1, oD, oW, oH, C))
    return out


# speedup vs baseline: 7.1685x; 7.1685x over previous
import functools

import jax
import jax.numpy as jnp
from jax import lax
from jax.experimental import pallas as pl
from jax.experimental.pallas import tpu as pltpu
from jax.experimental.pallas import tpu_sc as plsc

_BEV_FEAT_SHAPE = (1, 1, 200, 200, 64)
_NC, _NS, _LANES = 2, 16, 16
_NW = _NC * _NS
_IC = 80
_CC = 50
_NBUF = 3


@functools.lru_cache(maxsize=None)
def _make_bevpool(n_cells, mx, C, p1, f1):
    del p1, f1
    kc = C // _LANES
    G = _CC * mx
    n_str = G // _IC
    cells_per_w = n_cells // _NW
    n_chunks = cells_per_w // _CC
    mesh = plsc.VectorSubcoreMesh(
        core_axis_name="c", subcore_axis_name="s",
        num_cores=_NC, num_subcores=_NS)

    n_idx = cells_per_w * mx
    scratch = ([pltpu.VMEM((n_idx,), jnp.int32)] * 2
               + [pltpu.VMEM((G,), jnp.float32)] * _NBUF
               + [pltpu.VMEM((G, C), jnp.bfloat16)] * _NBUF
               + [pltpu.VMEM((_CC * C,), jnp.float32)] * _NBUF
               + [pltpu.SemaphoreType.DMA] * 2)

    @functools.partial(
        pl.kernel,
        out_type=jax.ShapeDtypeStruct((n_cells * C,), jnp.float32),
        mesh=mesh,
        compiler_params=pltpu.CompilerParams(
            use_tc_tiling_on_sc=False, needs_layout_passes=False),
        scratch_types=scratch,
    )
    def bev_kernel(depth_hbm, feat_hbm, rd_hbm, rf_hbm, out_hbm,
                   rdx_v, rfx_v, dv0, dv1, dv2, fr0, fr1, fr2,
                   ob0, ob1, ob2, sem_g, sem_o):
        dvs = (dv0, dv1, dv2)
        frs = (fr0, fr1, fr2)
        obs = (ob0, ob1, ob2)
        wid = lax.axis_index("s") * _NC + lax.axis_index("c")
        cell_base = wid * cells_per_w
        idx_base = wid * n_idx

        pltpu.sync_copy(rd_hbm.at[pl.ds(idx_base, n_idx)], rdx_v)
        pltpu.sync_copy(rf_hbm.at[pl.ds(idx_base, n_idx)], rfx_v)

        def fire_gathers(t, dv, fr):
            for i in range(n_str):
                r = (t * n_str + i) * _IC
                pltpu.async_copy(
                    depth_hbm.at[rdx_v.at[pl.ds(r, _IC)]],
                    dv.at[pl.ds(i * _IC, _IC)], sem_g)
                pltpu.async_copy(
                    feat_hbm.at[rfx_v.at[pl.ds(r, _IC)]],
                    fr.at[pl.ds(i * _IC, _IC)], sem_g)

        def drain_gathers(dv, fr):
            for i in range(n_str):
                pltpu.make_async_copy(
                    depth_hbm.at[rdx_v.at[pl.ds(i * _IC, _IC)]],
                    dv.at[pl.ds(i * _IC, _IC)], sem_g).wait()
                pltpu.make_async_copy(
                    feat_hbm.at[rfx_v.at[pl.ds(i * _IC, _IC)]],
                    fr.at[pl.ds(i * _IC, _IC)], sem_g).wait()

        def compute_chunk(t, dv, fr, ob):
            def cell_body(j, _):
                pb = j * mx
                dvec = dv[pl.ds(pb, mx)]
                dd = plsc.bitcast(
                    plsc.pack(dvec, dvec, format=plsc.PackFormat.INTERLEAVED),
                    jnp.int32)
                acc = [jnp.zeros((_LANES,), jnp.float32) for _ in range(kc)]
                for p in range(mx):
                    dsplat = plsc.bitcast(
                        jnp.broadcast_to(dd[p], (_LANES,)), jnp.bfloat16)
                    for k2 in range(kc // 2):
                        packed = fr[pb + p, pl.ds(k2 * 2 * _LANES, 2 * _LANES)]
                        lo, hi = plsc.unpack(
                            dsplat * packed, format=plsc.PackFormat.INTERLEAVED)
                        acc[2 * k2] = acc[2 * k2] + lo
                        acc[2 * k2 + 1] = acc[2 * k2 + 1] + hi
                for k in range(kc):
                    ob[pl.ds(j * C + k * _LANES, _LANES)] = acc[k]
                return 0

            lax.fori_loop(0, _CC, cell_body, 0)
            pltpu.async_copy(
                ob, out_hbm.at[pl.ds((cell_base + t * _CC) * C, _CC * C)],
                sem_o)

        def drain_out(ob):
            pltpu.make_async_copy(
                ob, out_hbm.at[pl.ds(cell_base * C, _CC * C)], sem_o).wait()

        fire_gathers(0, dvs[0], frs[0])
        fire_gathers(1, dvs[1], frs[1])

        n_main = n_chunks - 4

        def ring_body(q, _):
            for u in range(_NBUF):
                t = _NBUF * q + u
                fire_gathers(t + 2, dvs[(u + 2) % _NBUF], frs[(u + 2) % _NBUF])
                drain_gathers(dvs[u], frs[u])

                @pl.when(t >= _NBUF)
                def _():
                    drain_out(obs[u])

                compute_chunk(t, dvs[u], frs[u], obs[u])
            return 0

        lax.fori_loop(0, n_main // _NBUF, ring_body, 0)
        for tt in range(n_main, n_chunks):
            u = tt % _NBUF
            if tt + 2 < n_chunks:
                fire_gathers(tt + 2, dvs[(u + 2) % _NBUF], frs[(u + 2) % _NBUF])
            drain_gathers(dvs[u], frs[u])
            drain_out(obs[u])
            compute_chunk(tt, dvs[u], frs[u], obs[u])
        for tt in range(n_chunks - _NBUF, n_chunks):
            drain_out(obs[tt % _NBUF])

    return bev_kernel


def kernel(depth, feat, ranks_depth, ranks_feat, maxn):
    del maxn
    C = feat.shape[-1]
    _, oD, oW, oH, _ = _BEV_FEAT_SHAPE
    n_cells = oD * oW * oH
    L = ranks_depth.shape[0]
    mx = L // n_cells
    npad = 128 - depth.size % 128
    depth_flat = jnp.concatenate(
        [depth.reshape(-1), jnp.zeros((npad,), jnp.float32)])
    feat_2d = jnp.concatenate(
        [feat.reshape(-1, C), jnp.zeros((1, C), jnp.float32)], axis=0)
    f1 = feat_2d.shape[0]
    feat_bf = (feat_2d.reshape(f1, C // 32, 2, 16)
               .swapaxes(2, 3).reshape(f1, C).astype(jnp.bfloat16))
    fn = _make_bevpool(n_cells, mx, C, depth_flat.shape[0], f1)
    out = fn(depth_flat, feat_bf, ranks_depth, ranks_feat)
    return out.reshape(1, oD, oW, oH, C)
